# Initial kernel scaffold; baseline (speedup 1.0000x reference)
import functools

import jax
import jax.numpy as jnp
from jax import lax
from jax.experimental import pallas as pl
from jax.experimental.pallas import tpu as pltpu
from jax.experimental.pallas import tpu_sc as plsc

FEATURE_NUM = 100000
FIELD_NUM = 26
LATENT_NUM = 32
BATCH = 4096

D = FIELD_NUM * LATENT_NUM
NC = 2
NS = 16
NW = NC * NS
B_TOTAL = BATCH * FIELD_NUM
BPW = B_TOTAL // NW
CHUNK = 64
NCHUNK = BPW // CHUNK

_mesh = plsc.VectorSubcoreMesh(
    core_axis_name="c", subcore_axis_name="s", num_cores=NC, num_subcores=NS
)


@functools.partial(
    pl.kernel,
    out_type=jax.ShapeDtypeStruct((B_TOTAL, D), jnp.float32),
    mesh=_mesh,
    scratch_types=[
        pltpu.VMEM((BPW,), jnp.int32),
        pltpu.VMEM((CHUNK, D), jnp.float32),
        pltpu.SemaphoreType.DMA,
    ],
)
def _gather_kernel(table_hbm, idx_hbm, out_hbm, idx_v, rows_v, gsem):
    wid = lax.axis_index("s") * NC + lax.axis_index("c")
    base = wid * BPW
    pltpu.sync_copy(idx_hbm.at[pl.ds(base, BPW)], idx_v)

    def body(c, carry):
        off = pl.multiple_of(c * CHUNK, CHUNK)
        pltpu.async_copy(
            table_hbm.at[idx_v.at[pl.ds(off, CHUNK)]], rows_v, gsem
        ).wait()
        pltpu.sync_copy(rows_v, out_hbm.at[pl.ds(base + off, CHUNK)])
        return carry

    lax.fori_loop(0, NCHUNK, body, 0)


def kernel(x, weights):
    table = weights.reshape(FEATURE_NUM, D)
    idx = x.reshape(B_TOTAL)
    out = _gather_kernel(table, idx)
    return out.reshape(BATCH, FIELD_NUM, FIELD_NUM, LATENT_NUM)


# speedup vs baseline: 7.3753x; 7.3753x over previous
import functools

import jax
import jax.numpy as jnp
from jax import lax
from jax.experimental import pallas as pl
from jax.experimental.pallas import tpu as pltpu
from jax.experimental.pallas import tpu_sc as plsc

FEATURE_NUM = 100000
FIELD_NUM = 26
LATENT_NUM = 32
BATCH = 4096

D = FIELD_NUM * LATENT_NUM
NC = 2
NS = 16
NW = NC * NS
B_TOTAL = BATCH * FIELD_NUM
BPW = B_TOTAL // NW
CHUNK = 64
NCHUNK = BPW // CHUNK

_mesh = plsc.VectorSubcoreMesh(
    core_axis_name="c", subcore_axis_name="s", num_cores=NC, num_subcores=NS
)


@functools.partial(
    pl.kernel,
    out_type=jax.ShapeDtypeStruct((B_TOTAL, D), jnp.float32),
    mesh=_mesh,
    scratch_types=[
        pltpu.VMEM((BPW,), jnp.int32),
        pltpu.VMEM((CHUNK, D), jnp.float32),
        pltpu.SemaphoreType.DMA,
    ],
    compiler_params=pltpu.CompilerParams(use_tc_tiling_on_sc=False),
)
def _gather_kernel(table_hbm, idx_hbm, out_hbm, idx_v, rows_v, gsem):
    wid = lax.axis_index("s") * NC + lax.axis_index("c")
    base = wid * BPW
    pltpu.sync_copy(idx_hbm.at[pl.ds(base, BPW)], idx_v)

    def body(c, carry):
        off = pl.multiple_of(c * CHUNK, CHUNK)
        pltpu.async_copy(
            table_hbm.at[idx_v.at[pl.ds(off, CHUNK)]], rows_v, gsem
        ).wait()
        pltpu.sync_copy(rows_v, out_hbm.at[pl.ds(base + off, CHUNK)])
        return carry

    lax.fori_loop(0, NCHUNK, body, 0)


def kernel(x, weights):
    table = weights.reshape(FEATURE_NUM, D)
    idx = x.reshape(B_TOTAL)
    out = _gather_kernel(table, idx)
    return out.reshape(BATCH, FIELD_NUM, FIELD_NUM, LATENT_NUM)


# speedup vs baseline: 7.4383x; 1.0085x over previous
import functools

import jax
import jax.numpy as jnp
from jax import lax
from jax.experimental import pallas as pl
from jax.experimental.pallas import tpu as pltpu
from jax.experimental.pallas import tpu_sc as plsc

FEATURE_NUM = 100000
FIELD_NUM = 26
LATENT_NUM = 32
BATCH = 4096

D = FIELD_NUM * LATENT_NUM
NC = 2
NS = 16
NW = NC * NS
B_TOTAL = BATCH * FIELD_NUM
BPW = B_TOTAL // NW
CHUNK = 64
NCHUNK = BPW // CHUNK
NPAIR = NCHUNK // 2

_mesh = plsc.VectorSubcoreMesh(
    core_axis_name="c", subcore_axis_name="s", num_cores=NC, num_subcores=NS
)


@functools.partial(
    pl.kernel,
    out_type=jax.ShapeDtypeStruct((B_TOTAL, D), jnp.float32),
    mesh=_mesh,
    scratch_types=[
        pltpu.VMEM((BPW,), jnp.int32),
        pltpu.VMEM((CHUNK, D), jnp.float32),
        pltpu.VMEM((CHUNK, D), jnp.float32),
        pltpu.SemaphoreType.DMA,
        pltpu.SemaphoreType.DMA,
        pltpu.SemaphoreType.DMA,
        pltpu.SemaphoreType.DMA,
    ],
    compiler_params=pltpu.CompilerParams(use_tc_tiling_on_sc=False),
)
def _gather_kernel(table_hbm, idx_hbm, out_hbm, idx_v, buf0, buf1, g0, g1, s0, s1):
    wid = lax.axis_index("s") * NC + lax.axis_index("c")
    base = wid * BPW
    pltpu.sync_copy(idx_hbm.at[pl.ds(base, BPW)], idx_v)

    def start_gather(i, buf, sem):
        off = pl.multiple_of(i * CHUNK, CHUNK)
        pltpu.async_copy(table_hbm.at[idx_v.at[pl.ds(off, CHUNK)]], buf, sem)

    def start_store(i, buf, sem):
        off = pl.multiple_of(i * CHUNK, CHUNK)
        pltpu.async_copy(buf, out_hbm.at[pl.ds(base + off, CHUNK)], sem)

    def wait_gather(buf, sem):
        pltpu.make_async_copy(table_hbm.at[pl.ds(0, CHUNK)], buf, sem).wait()

    def wait_store(buf, sem):
        pltpu.make_async_copy(buf, out_hbm.at[pl.ds(base, CHUNK)], sem).wait()

    start_gather(0, buf0, g0)

    @pl.loop(0, NPAIR)
    def pair(p):
        i0 = p * 2

        @pl.when(p > 0)
        def _():
            wait_store(buf1, s1)

        start_gather(i0 + 1, buf1, g1)
        wait_gather(buf0, g0)
        start_store(i0, buf0, s0)

        @pl.when(p < NPAIR - 1)
        def _():
            wait_store(buf0, s0)
            start_gather(i0 + 2, buf0, g0)

        wait_gather(buf1, g1)
        start_store(i0 + 1, buf1, s1)

    wait_store(buf0, s0)
    wait_store(buf1, s1)


def kernel(x, weights):
    table = weights.reshape(FEATURE_NUM, D)
    idx = x.reshape(B_TOTAL)
    out = _gather_kernel(table, idx)
    return out.reshape(BATCH, FIELD_NUM, FIELD_NUM, LATENT_NUM)


# speedup vs baseline: 8.5914x; 1.1550x over previous
import functools

import jax
import jax.numpy as jnp
from jax import lax
from jax.experimental import pallas as pl
from jax.experimental.pallas import tpu as pltpu
from jax.experimental.pallas import tpu_sc as plsc

FEATURE_NUM = 100000
FIELD_NUM = 26
LATENT_NUM = 32
BATCH = 4096

NC = 2
NS = 16
NW = NC * NS
B_TOTAL = BATCH * FIELD_NUM
BPW = B_TOTAL // NW
CHUNK = 64
NCHUNK = BPW // CHUNK
NPAIR = NCHUNK // 2

_mesh = plsc.VectorSubcoreMesh(
    core_axis_name="c", subcore_axis_name="s", num_cores=NC, num_subcores=NS
)


@functools.partial(
    pl.kernel,
    out_type=jax.ShapeDtypeStruct((B_TOTAL, FIELD_NUM, LATENT_NUM), jnp.float32),
    mesh=_mesh,
    scratch_types=[
        pltpu.VMEM((BPW,), jnp.int32),
        pltpu.VMEM((CHUNK, FIELD_NUM, LATENT_NUM), jnp.float32),
        pltpu.VMEM((CHUNK, FIELD_NUM, LATENT_NUM), jnp.float32),
        pltpu.SemaphoreType.DMA,
        pltpu.SemaphoreType.DMA,
        pltpu.SemaphoreType.DMA,
        pltpu.SemaphoreType.DMA,
    ],
    compiler_params=pltpu.CompilerParams(use_tc_tiling_on_sc=False),
)
def _gather_kernel(table_hbm, idx_hbm, out_hbm, idx_v, buf0, buf1, g0, g1, s0, s1):
    wid = lax.axis_index("s") * NC + lax.axis_index("c")
    base = wid * BPW
    pltpu.sync_copy(idx_hbm.at[pl.ds(base, BPW)], idx_v)

    def start_gather(i, buf, sem):
        off = pl.multiple_of(i * CHUNK, CHUNK)
        pltpu.async_copy(table_hbm.at[idx_v.at[pl.ds(off, CHUNK)]], buf, sem)

    def start_store(i, buf, sem):
        off = pl.multiple_of(i * CHUNK, CHUNK)
        pltpu.async_copy(buf, out_hbm.at[pl.ds(base + off, CHUNK)], sem)

    def wait_gather(buf, sem):
        pltpu.make_async_copy(table_hbm.at[pl.ds(0, CHUNK)], buf, sem).wait()

    def wait_store(buf, sem):
        pltpu.make_async_copy(buf, out_hbm.at[pl.ds(base, CHUNK)], sem).wait()

    start_gather(0, buf0, g0)

    @pl.loop(0, NPAIR)
    def pair(p):
        i0 = p * 2

        @pl.when(p > 0)
        def _():
            wait_store(buf1, s1)

        start_gather(i0 + 1, buf1, g1)
        wait_gather(buf0, g0)
        start_store(i0, buf0, s0)

        @pl.when(p < NPAIR - 1)
        def _():
            wait_store(buf0, s0)
            start_gather(i0 + 2, buf0, g0)

        wait_gather(buf1, g1)
        start_store(i0 + 1, buf1, s1)

    wait_store(buf0, s0)
    wait_store(buf1, s1)


def kernel(x, weights):
    idx = x.reshape(B_TOTAL)
    out = _gather_kernel(weights, idx)
    return out.reshape(BATCH, FIELD_NUM, FIELD_NUM, LATENT_NUM)


# speedup vs baseline: 8.5967x; 1.0006x over previous
import functools

import jax
import jax.numpy as jnp
from jax import lax
from jax.experimental import pallas as pl
from jax.experimental.pallas import tpu as pltpu
from jax.experimental.pallas import tpu_sc as plsc

FEATURE_NUM = 100000
FIELD_NUM = 26
LATENT_NUM = 32
BATCH = 4096

NC = 2
NS = 16
NW = NC * NS
BPW = BATCH // NW
NPAIR = BPW // 2

_mesh = plsc.VectorSubcoreMesh(
    core_axis_name="c", subcore_axis_name="s", num_cores=NC, num_subcores=NS
)

_BLOCK = (FIELD_NUM, FIELD_NUM, LATENT_NUM)


@functools.partial(
    pl.kernel,
    out_type=jax.ShapeDtypeStruct((BATCH,) + _BLOCK, jnp.float32),
    mesh=_mesh,
    scratch_types=[
        pltpu.VMEM((BPW, FIELD_NUM), jnp.int32),
        pltpu.VMEM(_BLOCK, jnp.float32),
        pltpu.VMEM(_BLOCK, jnp.float32),
        pltpu.SemaphoreType.DMA,
        pltpu.SemaphoreType.DMA,
        pltpu.SemaphoreType.DMA,
        pltpu.SemaphoreType.DMA,
    ],
    compiler_params=pltpu.CompilerParams(use_tc_tiling_on_sc=False),
)
def _gather_kernel(table_hbm, x_hbm, out_hbm, idx_v, buf0, buf1, g0, g1, s0, s1):
    wid = lax.axis_index("s") * NC + lax.axis_index("c")
    base = wid * BPW
    pltpu.sync_copy(x_hbm.at[pl.ds(base, BPW)], idx_v)

    def start_gather(i, buf, sem):
        pltpu.async_copy(table_hbm.at[idx_v.at[i]], buf, sem)

    def start_store(i, buf, sem):
        pltpu.async_copy(buf, out_hbm.at[base + i], sem)

    def wait_gather(buf, sem):
        pltpu.make_async_copy(table_hbm.at[pl.ds(0, FIELD_NUM)], buf, sem).wait()

    def wait_store(buf, sem):
        pltpu.make_async_copy(buf, out_hbm.at[base], sem).wait()

    start_gather(0, buf0, g0)

    @pl.loop(0, NPAIR)
    def pair(p):
        i0 = p * 2

        @pl.when(p > 0)
        def _():
            wait_store(buf1, s1)

        start_gather(i0 + 1, buf1, g1)
        wait_gather(buf0, g0)
        start_store(i0, buf0, s0)

        @pl.when(p < NPAIR - 1)
        def _():
            wait_store(buf0, s0)
            start_gather(i0 + 2, buf0, g0)

        wait_gather(buf1, g1)
        start_store(i0 + 1, buf1, s1)

    wait_store(buf0, s0)
    wait_store(buf1, s1)


def kernel(x, weights):
    return _gather_kernel(weights, x)
